# baseline (device time: 28396 ns/iter reference)
import jax
import jax.numpy as jnp
from jax import lax
from jax.experimental import pallas as pl
from jax.experimental.pallas import tpu as pltpu

M = 2048
D = 1024
HALF = M // 2
SUB = HALF // 2

CHUNKS = (128, 96, 80, 64, 48, 48, 32, 16)
assert sum(CHUNKS) == SUB
NC = len(CHUNKS)
OFFS = tuple(sum(CHUNKS[:i]) for i in range(NC))


def kernel(partial, gamma):
    x2d = partial.reshape(M, D)
    g2d = gamma.reshape(1, D)

    def body(x_hbm, g_ref, out_ref, pbuf, mybuf, xsend, xrecv, ysend, yrecv,
             p_sems, m_sem, xs_sems, xr_sems, ys_sems, yr_sems):
        my_x = lax.axis_index("x")
        my_y = lax.axis_index("y")
        my_z = lax.axis_index("z")
        x_peer = (1 - my_x, my_y, my_z)
        y_peer = (my_x, 1 - my_y, my_z)

        peer_rows = (1 - my_x) * HALF + my_y * SUB
        my_rows = my_x * HALF + my_y * SUB

        p_dmas = []
        for i in range(NC):
            d = pltpu.make_async_copy(
                x_hbm.at[pl.ds(peer_rows + OFFS[i], CHUNKS[i])],
                pbuf.at[pl.ds(OFFS[i], CHUNKS[i])],
                p_sems.at[i],
            )
            d.start()
            p_dmas.append(d)
        m_dma = pltpu.make_async_copy(
            x_hbm.at[pl.ds(my_rows, SUB)], mybuf, m_sem)
        m_dma.start()

        barrier = pltpu.get_barrier_semaphore()
        for peer in (x_peer, y_peer):
            pl.semaphore_signal(
                barrier, inc=1, device_id=peer,
                device_id_type=pl.DeviceIdType.MESH,
            )
        pl.semaphore_wait(barrier, 2)

        x_rdmas = []
        for i in range(NC):
            p_dmas[i].wait()
            xsend[pl.ds(OFFS[i], CHUNKS[i]), :] = (
                pbuf[pl.ds(OFFS[i], CHUNKS[i]), :].astype(jnp.bfloat16))
            r = pltpu.make_async_remote_copy(
                src_ref=xsend.at[pl.ds(OFFS[i], CHUNKS[i])],
                dst_ref=xrecv.at[pl.ds(OFFS[i], CHUNKS[i])],
                send_sem=xs_sems.at[i], recv_sem=xr_sems.at[i],
                device_id=x_peer, device_id_type=pl.DeviceIdType.MESH,
            )
            r.start()
            x_rdmas.append(r)

        m_dma.wait()

        LAG = 2
        y_rdmas = []

        def drain_y(i):
            y_rdmas[i].wait_recv()
            out_ref[pl.ds((1 - my_y) * SUB + OFFS[i], CHUNKS[i]), :] = (
                yrecv[pl.ds(OFFS[i], CHUNKS[i]), :].astype(jnp.float32))

        for i in range(NC):
            x_rdmas[i].wait_recv()
            acc = (mybuf[pl.ds(OFFS[i], CHUNKS[i]), :]
                   + xrecv[pl.ds(OFFS[i], CHUNKS[i]), :].astype(jnp.float32))
            ms = jnp.mean(acc * acc, axis=-1, keepdims=True)
            normed = acc * lax.rsqrt(ms + 1e-6) * g_ref[...]
            out_ref[pl.ds(my_y * SUB + OFFS[i], CHUNKS[i]), :] = normed
            ysend[pl.ds(OFFS[i], CHUNKS[i]), :] = normed.astype(jnp.bfloat16)
            r = pltpu.make_async_remote_copy(
                src_ref=ysend.at[pl.ds(OFFS[i], CHUNKS[i])],
                dst_ref=yrecv.at[pl.ds(OFFS[i], CHUNKS[i])],
                send_sem=ys_sems.at[i], recv_sem=yr_sems.at[i],
                device_id=y_peer, device_id_type=pl.DeviceIdType.MESH,
            )
            r.start()
            y_rdmas.append(r)
            if i >= LAG:
                drain_y(i - LAG)

        for i in range(NC - LAG, NC):
            drain_y(i)

        for i in range(NC):
            x_rdmas[i].wait_send()
            y_rdmas[i].wait_send()

    return pl.pallas_call(
        body,
        out_shape=jax.ShapeDtypeStruct((HALF, D), jnp.float32),
        in_specs=[
            pl.BlockSpec(memory_space=pl.ANY),
            pl.BlockSpec(memory_space=pltpu.VMEM),
        ],
        out_specs=pl.BlockSpec(memory_space=pltpu.VMEM),
        scratch_shapes=[
            pltpu.VMEM((SUB, D), jnp.float32),
            pltpu.VMEM((SUB, D), jnp.float32),
            pltpu.VMEM((SUB, D), jnp.bfloat16),
            pltpu.VMEM((SUB, D), jnp.bfloat16),
            pltpu.VMEM((SUB, D), jnp.bfloat16),
            pltpu.VMEM((SUB, D), jnp.bfloat16),
            pltpu.SemaphoreType.DMA((NC,)),
            pltpu.SemaphoreType.DMA,
            pltpu.SemaphoreType.DMA((NC,)),
            pltpu.SemaphoreType.DMA((NC,)),
            pltpu.SemaphoreType.DMA((NC,)),
            pltpu.SemaphoreType.DMA((NC,)),
        ],
        compiler_params=pltpu.CompilerParams(collective_id=0),
    )(x2d, g2d)


# device time: 27433 ns/iter; 1.0351x vs baseline; 1.0351x over previous
import jax
import jax.numpy as jnp
from jax import lax
from jax.experimental import pallas as pl
from jax.experimental.pallas import tpu as pltpu

M = 2048
D = 1024
HALF = M // 2
SUB = HALF // 2

CHUNKS = (128, 96, 80, 64, 48, 48, 32, 16)
assert sum(CHUNKS) == SUB
NC = len(CHUNKS)
OFFS = tuple(sum(CHUNKS[:i]) for i in range(NC))


def kernel(partial, gamma):
    x2d = partial.reshape(M, D)
    g2d = gamma.reshape(1, D)

    def body(x_hbm, g_ref, out_ref, pbuf, mybuf, xsend, xrecv, ysend, yrecv,
             p_sems, m_sem, xs_sems, xr_sems, ys_sems, yr_sems):
        my_x = lax.axis_index("x")
        my_y = lax.axis_index("y")
        my_z = lax.axis_index("z")
        x_peer = (1 - my_x, my_y, my_z)
        y_peer = (my_x, 1 - my_y, my_z)

        peer_rows = (1 - my_x) * HALF + my_y * SUB
        my_rows = my_x * HALF + my_y * SUB

        p_dma = pltpu.make_async_copy(
            x_hbm.at[pl.ds(peer_rows, SUB)], pbuf, p_sems.at[0])
        p_dma.start()
        m_dma = pltpu.make_async_copy(
            x_hbm.at[pl.ds(my_rows, SUB)], mybuf, m_sem)
        m_dma.start()

        barrier = pltpu.get_barrier_semaphore()
        for peer in (x_peer, y_peer):
            pl.semaphore_signal(
                barrier, inc=1, device_id=peer,
                device_id_type=pl.DeviceIdType.MESH,
            )
        pl.semaphore_wait(barrier, 2)

        p_dma.wait()
        x_rdmas = []
        for i in range(NC):
            xsend[pl.ds(OFFS[i], CHUNKS[i]), :] = (
                pbuf[pl.ds(OFFS[i], CHUNKS[i]), :].astype(jnp.bfloat16))
            r = pltpu.make_async_remote_copy(
                src_ref=xsend.at[pl.ds(OFFS[i], CHUNKS[i])],
                dst_ref=xrecv.at[pl.ds(OFFS[i], CHUNKS[i])],
                send_sem=xs_sems.at[i], recv_sem=xr_sems.at[i],
                device_id=x_peer, device_id_type=pl.DeviceIdType.MESH,
            )
            r.start()
            x_rdmas.append(r)

        m_dma.wait()

        LAG = 2
        y_rdmas = []

        def drain_y(i):
            y_rdmas[i].wait_recv()
            out_ref[pl.ds((1 - my_y) * SUB + OFFS[i], CHUNKS[i]), :] = (
                yrecv[pl.ds(OFFS[i], CHUNKS[i]), :].astype(jnp.float32))

        for i in range(NC):
            x_rdmas[i].wait_recv()
            acc = (mybuf[pl.ds(OFFS[i], CHUNKS[i]), :]
                   + xrecv[pl.ds(OFFS[i], CHUNKS[i]), :].astype(jnp.float32))
            ms = jnp.mean(acc * acc, axis=-1, keepdims=True)
            normed = acc * lax.rsqrt(ms + 1e-6) * g_ref[...]
            out_ref[pl.ds(my_y * SUB + OFFS[i], CHUNKS[i]), :] = normed
            ysend[pl.ds(OFFS[i], CHUNKS[i]), :] = normed.astype(jnp.bfloat16)
            r = pltpu.make_async_remote_copy(
                src_ref=ysend.at[pl.ds(OFFS[i], CHUNKS[i])],
                dst_ref=yrecv.at[pl.ds(OFFS[i], CHUNKS[i])],
                send_sem=ys_sems.at[i], recv_sem=yr_sems.at[i],
                device_id=y_peer, device_id_type=pl.DeviceIdType.MESH,
            )
            r.start()
            y_rdmas.append(r)
            if i >= LAG:
                drain_y(i - LAG)

        for i in range(NC - LAG, NC):
            drain_y(i)

        for i in range(NC):
            x_rdmas[i].wait_send()
            y_rdmas[i].wait_send()

    return pl.pallas_call(
        body,
        out_shape=jax.ShapeDtypeStruct((HALF, D), jnp.float32),
        in_specs=[
            pl.BlockSpec(memory_space=pl.ANY),
            pl.BlockSpec(memory_space=pltpu.VMEM),
        ],
        out_specs=pl.BlockSpec(memory_space=pltpu.VMEM),
        scratch_shapes=[
            pltpu.VMEM((SUB, D), jnp.float32),
            pltpu.VMEM((SUB, D), jnp.float32),
            pltpu.VMEM((SUB, D), jnp.bfloat16),
            pltpu.VMEM((SUB, D), jnp.bfloat16),
            pltpu.VMEM((SUB, D), jnp.bfloat16),
            pltpu.VMEM((SUB, D), jnp.bfloat16),
            pltpu.SemaphoreType.DMA((NC,)),
            pltpu.SemaphoreType.DMA,
            pltpu.SemaphoreType.DMA((NC,)),
            pltpu.SemaphoreType.DMA((NC,)),
            pltpu.SemaphoreType.DMA((NC,)),
            pltpu.SemaphoreType.DMA((NC,)),
        ],
        compiler_params=pltpu.CompilerParams(collective_id=0),
    )(x2d, g2d)


# device time: 27420 ns/iter; 1.0356x vs baseline; 1.0005x over previous
import jax
import jax.numpy as jnp
from jax import lax
from jax.experimental import pallas as pl
from jax.experimental.pallas import tpu as pltpu

M = 2048
D = 1024
HALF = M // 2
SUB = HALF // 2

CHUNKS = (128, 96, 80, 64, 48, 48, 32, 16)
assert sum(CHUNKS) == SUB
NC = len(CHUNKS)
OFFS = tuple(sum(CHUNKS[:i]) for i in range(NC))


def kernel(partial, gamma):
    def body(x_hbm, g_ref, out_ref, pbuf, mybuf, xsend, xrecv, ysend, yrecv,
             p_sems, m_sem, xs_sems, xr_sems, ys_sems, yr_sems):
        my_x = lax.axis_index("x")
        my_y = lax.axis_index("y")
        my_z = lax.axis_index("z")
        x_peer = (1 - my_x, my_y, my_z)
        y_peer = (my_x, 1 - my_y, my_z)

        peer_rows = (1 - my_x) * HALF + my_y * SUB
        my_rows = my_x * HALF + my_y * SUB

        p_dma = pltpu.make_async_copy(
            x_hbm.at[0, pl.ds(peer_rows, SUB)], pbuf, p_sems.at[0])
        p_dma.start()
        m_dma = pltpu.make_async_copy(
            x_hbm.at[0, pl.ds(my_rows, SUB)], mybuf, m_sem)
        m_dma.start()

        barrier = pltpu.get_barrier_semaphore()
        for peer in (x_peer, y_peer):
            pl.semaphore_signal(
                barrier, inc=1, device_id=peer,
                device_id_type=pl.DeviceIdType.MESH,
            )
        pl.semaphore_wait(barrier, 2)

        gam = g_ref[...].reshape(1, D)

        p_dma.wait()
        x_rdmas = []
        for i in range(NC):
            xsend[pl.ds(OFFS[i], CHUNKS[i]), :] = (
                pbuf[pl.ds(OFFS[i], CHUNKS[i]), :].astype(jnp.bfloat16))
            r = pltpu.make_async_remote_copy(
                src_ref=xsend.at[pl.ds(OFFS[i], CHUNKS[i])],
                dst_ref=xrecv.at[pl.ds(OFFS[i], CHUNKS[i])],
                send_sem=xs_sems.at[i], recv_sem=xr_sems.at[i],
                device_id=x_peer, device_id_type=pl.DeviceIdType.MESH,
            )
            r.start()
            x_rdmas.append(r)

        m_dma.wait()

        LAG = 2
        y_rdmas = []

        def drain_y(i):
            y_rdmas[i].wait_recv()
            out_ref[pl.ds((1 - my_y) * SUB + OFFS[i], CHUNKS[i]), :] = (
                yrecv[pl.ds(OFFS[i], CHUNKS[i]), :].astype(jnp.float32))

        for i in range(NC):
            x_rdmas[i].wait_recv()
            acc = (mybuf[pl.ds(OFFS[i], CHUNKS[i]), :]
                   + xrecv[pl.ds(OFFS[i], CHUNKS[i]), :].astype(jnp.float32))
            ms = jnp.mean(acc * acc, axis=-1, keepdims=True)
            normed = acc * lax.rsqrt(ms + 1e-6) * gam
            out_ref[pl.ds(my_y * SUB + OFFS[i], CHUNKS[i]), :] = normed
            ysend[pl.ds(OFFS[i], CHUNKS[i]), :] = normed.astype(jnp.bfloat16)
            r = pltpu.make_async_remote_copy(
                src_ref=ysend.at[pl.ds(OFFS[i], CHUNKS[i])],
                dst_ref=yrecv.at[pl.ds(OFFS[i], CHUNKS[i])],
                send_sem=ys_sems.at[i], recv_sem=yr_sems.at[i],
                device_id=y_peer, device_id_type=pl.DeviceIdType.MESH,
            )
            r.start()
            y_rdmas.append(r)
            if i >= LAG:
                drain_y(i - LAG)

        for i in range(NC - LAG, NC):
            drain_y(i)

        for i in range(NC):
            x_rdmas[i].wait_send()
            y_rdmas[i].wait_send()

    return pl.pallas_call(
        body,
        out_shape=jax.ShapeDtypeStruct((HALF, D), jnp.float32),
        in_specs=[
            pl.BlockSpec(memory_space=pl.ANY),
            pl.BlockSpec(memory_space=pltpu.VMEM),
        ],
        out_specs=pl.BlockSpec(memory_space=pltpu.VMEM),
        scratch_shapes=[
            pltpu.VMEM((SUB, D), jnp.float32),
            pltpu.VMEM((SUB, D), jnp.float32),
            pltpu.VMEM((SUB, D), jnp.bfloat16),
            pltpu.VMEM((SUB, D), jnp.bfloat16),
            pltpu.VMEM((SUB, D), jnp.bfloat16),
            pltpu.VMEM((SUB, D), jnp.bfloat16),
            pltpu.SemaphoreType.DMA((NC,)),
            pltpu.SemaphoreType.DMA,
            pltpu.SemaphoreType.DMA((NC,)),
            pltpu.SemaphoreType.DMA((NC,)),
            pltpu.SemaphoreType.DMA((NC,)),
            pltpu.SemaphoreType.DMA((NC,)),
        ],
        compiler_params=pltpu.CompilerParams(collective_id=0),
    )(partial, gamma)


# device time: 22042 ns/iter; 1.2883x vs baseline; 1.2440x over previous
import jax
import jax.numpy as jnp
from jax import lax
from jax.experimental import pallas as pl
from jax.experimental.pallas import tpu as pltpu

M = 2048
D = 1024
HALF = M // 2
SUB = HALF // 2
NC = 8
CK = SUB // NC


def kernel(partial, gamma):
    def body(x_hbm, g_hbm, out_hbm, pbuf, mybuf, obuf, ybuf, gbuf,
             xsend, xrecv, ysend, yrecv,
             p_sems, m_sem, g_sem, o_sems, o2_sems,
             xs_sems, xr_sems, ys_sems, yr_sems):
        my_x = lax.axis_index("x")
        my_y = lax.axis_index("y")
        my_z = lax.axis_index("z")
        x_peer = (1 - my_x, my_y, my_z)
        y_peer = (my_x, 1 - my_y, my_z)

        peer_rows = (1 - my_x) * HALF + my_y * SUB
        my_rows = my_x * HALF + my_y * SUB

        p0_dma = pltpu.make_async_copy(
            x_hbm.at[0, pl.ds(peer_rows, CK)],
            pbuf.at[pl.ds(0, CK)], p_sems.at[0])
        p0_dma.start()
        p1_dma = pltpu.make_async_copy(
            x_hbm.at[0, pl.ds(peer_rows + CK, SUB - CK)],
            pbuf.at[pl.ds(CK, SUB - CK)], p_sems.at[1])
        p1_dma.start()
        m_dma = pltpu.make_async_copy(
            x_hbm.at[0, pl.ds(my_rows, SUB)], mybuf, m_sem)
        m_dma.start()
        g_dma = pltpu.make_async_copy(g_hbm, gbuf, g_sem)
        g_dma.start()

        barrier = pltpu.get_barrier_semaphore()
        for peer in (x_peer, y_peer):
            pl.semaphore_signal(
                barrier, inc=1, device_id=peer,
                device_id_type=pl.DeviceIdType.MESH,
            )
        pl.semaphore_wait(barrier, 2)


        x_rdmas = []
        for i in range(NC):
            if i == 0:
                p0_dma.wait()
            elif i == 1:
                p1_dma.wait()
            xsend[pl.ds(i * CK, CK), :] = (
                pbuf[pl.ds(i * CK, CK), :].astype(jnp.bfloat16))
            r = pltpu.make_async_remote_copy(
                src_ref=xsend.at[pl.ds(i * CK, CK)],
                dst_ref=xrecv.at[pl.ds(i * CK, CK)],
                send_sem=xs_sems.at[i], recv_sem=xr_sems.at[i],
                device_id=x_peer, device_id_type=pl.DeviceIdType.MESH,
            )
            r.start()
            x_rdmas.append(r)

        m_dma.wait()
        g_dma.wait()
        gam = gbuf[...].reshape(1, D)

        LAG = 2
        y_rdmas = []
        out_dmas = []

        def drain_y(i):
            y_rdmas[i].wait_recv()
            ybuf[pl.ds(i * CK, CK), :] = (
                yrecv[pl.ds(i * CK, CK), :].astype(jnp.float32))
            od = pltpu.make_async_copy(
                ybuf.at[pl.ds(i * CK, CK)],
                out_hbm.at[pl.ds((1 - my_y) * SUB + i * CK, CK)],
                o2_sems.at[i])
            od.start()
            out_dmas.append(od)

        for i in range(NC):
            x_rdmas[i].wait_recv()
            acc = (mybuf[pl.ds(i * CK, CK), :]
                   + xrecv[pl.ds(i * CK, CK), :].astype(jnp.float32))
            ms = jnp.mean(acc * acc, axis=-1, keepdims=True)
            normed = acc * lax.rsqrt(ms + 1e-6) * gam
            obuf[pl.ds(i * CK, CK), :] = normed
            od = pltpu.make_async_copy(
                obuf.at[pl.ds(i * CK, CK)],
                out_hbm.at[pl.ds(my_y * SUB + i * CK, CK)],
                o_sems.at[i])
            od.start()
            out_dmas.append(od)
            ysend[pl.ds(i * CK, CK), :] = normed.astype(jnp.bfloat16)
            r = pltpu.make_async_remote_copy(
                src_ref=ysend.at[pl.ds(i * CK, CK)],
                dst_ref=yrecv.at[pl.ds(i * CK, CK)],
                send_sem=ys_sems.at[i], recv_sem=yr_sems.at[i],
                device_id=y_peer, device_id_type=pl.DeviceIdType.MESH,
            )
            r.start()
            y_rdmas.append(r)
            if i >= LAG:
                drain_y(i - LAG)

        for i in range(NC - LAG, NC):
            drain_y(i)

        for od in out_dmas:
            od.wait()
        for i in range(NC):
            x_rdmas[i].wait_send()
            y_rdmas[i].wait_send()

    partial_hbm = pltpu.with_memory_space_constraint(
        partial, pltpu.MemorySpace.HBM)
    gamma_hbm = pltpu.with_memory_space_constraint(
        gamma, pltpu.MemorySpace.HBM)
    return pl.pallas_call(
        body,
        out_shape=jax.ShapeDtypeStruct((HALF, D), jnp.float32),
        in_specs=[
            pl.BlockSpec(memory_space=pltpu.MemorySpace.HBM),
            pl.BlockSpec(memory_space=pltpu.MemorySpace.HBM),
        ],
        out_specs=pl.BlockSpec(memory_space=pltpu.MemorySpace.HBM),
        scratch_shapes=[
            pltpu.VMEM((SUB, D), jnp.float32),
            pltpu.VMEM((SUB, D), jnp.float32),
            pltpu.VMEM((SUB, D), jnp.float32),
            pltpu.VMEM((SUB, D), jnp.float32),
            pltpu.VMEM((D,), jnp.float32),
            pltpu.VMEM((SUB, D), jnp.bfloat16),
            pltpu.VMEM((SUB, D), jnp.bfloat16),
            pltpu.VMEM((SUB, D), jnp.bfloat16),
            pltpu.VMEM((SUB, D), jnp.bfloat16),
            pltpu.SemaphoreType.DMA((2,)),
            pltpu.SemaphoreType.DMA,
            pltpu.SemaphoreType.DMA,
            pltpu.SemaphoreType.DMA((NC,)),
            pltpu.SemaphoreType.DMA((NC,)),
            pltpu.SemaphoreType.DMA((NC,)),
            pltpu.SemaphoreType.DMA((NC,)),
            pltpu.SemaphoreType.DMA((NC,)),
            pltpu.SemaphoreType.DMA((NC,)),
        ],
        compiler_params=pltpu.CompilerParams(collective_id=0),
    )(partial_hbm, gamma_hbm)
